# Initial kernel scaffold; baseline (speedup 1.0000x reference)
import jax
import jax.numpy as jnp
from jax.experimental import pallas as pl


def kernel(x, wg, w1, w2):
    raise NotImplementedError("write your pallas kernel here")


# speedup vs baseline: 1.7162x; 1.7162x over previous
import functools

import jax
import jax.numpy as jnp
from jax import lax
from jax.experimental import pallas as pl
from jax.experimental.pallas import tpu as pltpu
from jax.experimental.pallas import tpu_sc as plsc

E = 64
D = 768
FF = 1536
N = 32768
CAP = 512
EC = E * CAP
ZROW = EC

NB = 32
BN = N // NB

NC = 2
NS = 16
NW = NC * NS
TPW = N // NW
CH = 128
NCH = TPW // CH


def _gating_body(x_ref, wg_ref, xs_ref, idx_ref, laux_ref, cnt_ref, me_ref):
    i = pl.program_id(0)

    @pl.when(i == 0)
    def _():
        cnt_ref[...] = jnp.zeros_like(cnt_ref)
        me_ref[...] = jnp.zeros_like(me_ref)

    xb = x_ref[...]
    logits = jnp.dot(xb, wg_ref[...], preferred_element_type=jnp.float32)
    rowmax = jnp.max(logits, axis=1, keepdims=True)
    ex = jnp.exp(logits - rowmax)
    sumex = jnp.sum(ex, axis=1, keepdims=True)
    gate = 1.0 / sumex
    lane = lax.broadcasted_iota(jnp.int32, (BN, E), 1)
    am = jnp.min(jnp.where(logits == rowmax, lane, E), axis=1, keepdims=True)
    mask = (lane == am).astype(jnp.float32)

    r = lax.broadcasted_iota(jnp.int32, (BN, BN), 0)
    c = lax.broadcasted_iota(jnp.int32, (BN, BN), 1)
    tri = (c < r).astype(jnp.float32)
    loc = jnp.dot(tri, mask, preferred_element_type=jnp.float32) + cnt_ref[...]
    loc_s = jnp.sum(loc * mask, axis=1, keepdims=True)

    me_ref[...] += jnp.sum(ex / sumex, axis=0, keepdims=True)
    cnt_ref[...] += jnp.sum(mask, axis=0, keepdims=True)

    valid = loc_s < CAP
    slot = am * CAP + jnp.minimum(loc_s.astype(jnp.int32), CAP - 1)
    idx_ref[0, :, :] = jnp.where(valid, slot, ZROW)
    xs_ref[...] = xb * (gate * valid.astype(jnp.float32))

    @pl.when(i == NB - 1)
    def _():
        laux_ref[...] = jnp.full((1, 1), E / (N * N)) * jnp.sum(
            me_ref[...] * cnt_ref[...])


_gating = pl.pallas_call(
    _gating_body,
    grid=(NB,),
    in_specs=[
        pl.BlockSpec((BN, D), lambda i: (i, 0)),
        pl.BlockSpec((D, E), lambda i: (0, 0)),
    ],
    out_specs=[
        pl.BlockSpec((BN, D), lambda i: (i, 0)),
        pl.BlockSpec((1, BN, 1), lambda i: (i, 0, 0)),
        pl.BlockSpec((1, 1), lambda i: (0, 0)),
    ],
    out_shape=[
        jax.ShapeDtypeStruct((N, D), jnp.float32),
        jax.ShapeDtypeStruct((NB, BN, 1), jnp.int32),
        jax.ShapeDtypeStruct((1, 1), jnp.float32),
    ],
    scratch_shapes=[
        pltpu.VMEM((1, E), jnp.float32),
        pltpu.VMEM((1, E), jnp.float32),
    ],
    compiler_params=pltpu.CompilerParams(
        dimension_semantics=("arbitrary",),
    ),
)


def _ffn_body(disp_ref, w1_ref, w2_ref, out_ref):
    e = pl.program_id(0)

    @pl.when(e < E)
    def _():
        h = jnp.maximum(
            jnp.dot(disp_ref[...], w1_ref[0], preferred_element_type=jnp.float32),
            0.0)
        out_ref[...] = jnp.dot(h, w2_ref[0], preferred_element_type=jnp.float32)

    @pl.when(e == E)
    def _():
        out_ref[...] = jnp.zeros_like(out_ref)


_ffn = pl.pallas_call(
    _ffn_body,
    grid=(E + 1,),
    in_specs=[
        pl.BlockSpec((CAP, D), lambda e: (e, 0)),
        pl.BlockSpec((1, D, FF), lambda e: (jnp.minimum(e, E - 1), 0, 0)),
        pl.BlockSpec((1, FF, D), lambda e: (jnp.minimum(e, E - 1), 0, 0)),
    ],
    out_specs=pl.BlockSpec((CAP, D), lambda e: (e, 0)),
    out_shape=jax.ShapeDtypeStruct((EC + CAP, D), jnp.float32),
    compiler_params=pltpu.CompilerParams(
        dimension_semantics=("arbitrary",),
        vmem_limit_bytes=100 * 1024 * 1024,
    ),
)


@functools.lru_cache(maxsize=None)
def _sc_kernels():
    mesh = plsc.VectorSubcoreMesh(core_axis_name="c", subcore_axis_name="s")
    scratch = [
        pltpu.VMEM((NCH, CH), jnp.int32),
        pltpu.VMEM((CH, D), jnp.float32),
        pltpu.SemaphoreType.DMA,
    ]

    @functools.partial(
        pl.kernel,
        out_type=jax.ShapeDtypeStruct((EC + CAP, D), jnp.float32),
        mesh=mesh,
        scratch_types=scratch,
    )
    def dispatch(xs_hbm, idx_hbm, disp_hbm, idx_v, buf, sem):
        wid = lax.axis_index("s") * NC + lax.axis_index("c")
        pltpu.sync_copy(idx_hbm.at[wid], idx_v)
        for j in range(NCH):
            base = wid * TPW + j * CH
            pltpu.sync_copy(xs_hbm.at[pl.ds(base, CH)], buf)
            pltpu.async_copy(buf, disp_hbm.at[idx_v.at[j]], sem).wait()

    @functools.partial(
        pl.kernel,
        out_type=jax.ShapeDtypeStruct((N, D), jnp.float32),
        mesh=mesh,
        scratch_types=scratch,
    )
    def combine(eo_hbm, idx_hbm, y_hbm, idx_v, buf, sem):
        wid = lax.axis_index("s") * NC + lax.axis_index("c")
        pltpu.sync_copy(idx_hbm.at[wid], idx_v)
        for j in range(NCH):
            base = wid * TPW + j * CH
            pltpu.async_copy(eo_hbm.at[idx_v.at[j]], buf, sem).wait()
            pltpu.sync_copy(buf, y_hbm.at[pl.ds(base, CH)])

    return dispatch, combine


def kernel(x, wg, w1, w2):
    dispatch, combine = _sc_kernels()
    xs, idx3, laux = _gating(x, wg)
    idxr = idx3.reshape(NW, NCH, CH)
    disp = dispatch(xs, idxr)
    eo = _ffn(disp, w1, w2)
    y = combine(eo, idxr)
    return y, laux[0, 0]
